# Initial kernel scaffold; baseline (speedup 1.0000x reference)
import jax
import jax.numpy as jnp
from jax.experimental import pallas as pl


def kernel(x_pfas, x_gw, x_sw, eas, params, eis):
    raise NotImplementedError("write your pallas kernel here")


# speedup vs baseline: 1.9636x; 1.9636x over previous
import jax
import jax.numpy as jnp
from jax.experimental import pallas as pl


def _econv(p, x_src, x_dst, row, col, ea, cnt, win, has):
    n = x_dst.shape[0]
    agg = jax.ops.segment_sum(x_src[row], col, num_segments=n)
    mean = agg / jnp.maximum(cnt, 1.0)[:, None]
    P = mean @ p['W_l'].T + p['b_l'] + x_dst @ p['W_r'].T
    ea_w = ea[jnp.where(has, win, 0)]
    E = ea_w @ p['W_e'].T + p['b_e']
    Wg = p['W_g']
    G = jax.nn.sigmoid(P @ Wg[:, :128].T + E @ Wg[:, 128:].T + p['b_g'])
    out = P + jnp.where(has, 1.0, 0.0)[:, None] * (G * E)
    mu = out.mean(axis=0)
    var = out.var(axis=0)
    out = (out - mu) / jnp.sqrt(var + 1e-5) * p['gamma'] + p['beta']
    return jax.nn.relu(out + out)


def _self_sage(p, x_g):
    return x_g @ (p['W_l'] + p['W_r']).T + p['b_l']


def _hetero_opt(P, x_p, x_g, x_s, eis, eas, aux):
    out_g = (_econv(P['pg'], x_p, x_g, *aux['pg'])
             + _econv(P['gg'], x_g, x_g, *aux['gg'])
             + _self_sage(P['self'], x_g))
    out_p = (_econv(P['gp'], x_g, x_p, *aux['gp'])
             + _econv(P['sp'], x_s, x_p, *aux['sp']))
    out_s = _econv(P['ps'], x_p, x_s, *aux['ps'])
    return out_p, out_g, out_s


def kernel(x_pfas, x_gw, x_sw, eas, params, eis):
    n_dst = {'pg': x_gw.shape[0], 'gg': x_gw.shape[0],
             'gp': x_pfas.shape[0], 'sp': x_pfas.shape[0],
             'ps': x_sw.shape[0]}
    aux = {}
    for r in ('pg', 'gg', 'gp', 'sp', 'ps'):
        row, col = eis[r][0], eis[r][1]
        n = n_dst[r]
        e = row.shape[0]
        cnt = jax.ops.segment_sum(jnp.ones((e,), jnp.float32), col, num_segments=n)
        win = jax.ops.segment_max(jnp.arange(e, dtype=jnp.int32), col, num_segments=n)
        has = cnt > 0
        aux[r] = (row, col, eas[r], cnt, win, has)

    xp, xg, xs = _hetero_opt(params['l1'], x_pfas, x_gw, x_sw, eis, eas, aux)
    xp, xg, xs = jax.nn.relu(xp), jax.nn.relu(xg), jax.nn.relu(xs)
    xp, xg, xs = _hetero_opt(params['l2'], xp, xg, xs, eis, eas, aux)
    xp, xg, xs = jax.nn.relu(xp), jax.nn.relu(xg), jax.nn.relu(xs)
    xg = jnp.where(xg > 0, xg, params['prelu_gw'] * xg)
    xs = jnp.where(xs > 0, xs, params['prelu_sw'] * xs)
    xg = xg @ params['W_out_gw'].T + params['b_out_gw']
    xs = xs @ params['W_out_sw'].T + params['b_out_sw']
    return xp, xg, xs


# speedup vs baseline: 2.2086x; 1.1247x over previous
import functools

import jax
import jax.numpy as jnp
from jax import lax
from jax.experimental import pallas as pl
from jax.experimental.pallas import tpu as pltpu
from jax.experimental.pallas import tpu_sc as plsc

BN = 1000

_RELS = ('pg', 'gg', 'gp', 'sp', 'ps')
_SRCDST = {'pg': ('p', 'g'), 'gg': ('g', 'g'), 'gp': ('g', 'p'),
           'sp': ('s', 'p'), 'ps': ('p', 's')}


def _dotT(x, w):
    return lax.dot_general(x, w, (((1,), (1,)), ((), ())),
                           preferred_element_type=jnp.float32)


def _k1_body(agg_r, xd_r, cnt_r, eaw_r, wl_r, wr_r, we_r, wg1_r, wg2_r,
             aux_r, pre_r, st_r):
    i = pl.program_id(0)
    cnt = cnt_r[:, 0:1]
    has = (cnt > 0.0).astype(jnp.float32)
    inv = 1.0 / jnp.maximum(cnt, 1.0)
    mean = agg_r[...] * inv
    P = _dotT(mean, wl_r[...]) + aux_r[0:1, :] + _dotT(xd_r[...], wr_r[...])
    E = _dotT(eaw_r[...], we_r[...]) + aux_r[1:2, :]
    G = jax.nn.sigmoid(_dotT(P, wg1_r[...]) + _dotT(E, wg2_r[...])
                       + aux_r[2:3, :])
    pre = P + has * (G * E)
    pre_r[...] = pre

    @pl.when(i == 0)
    def _():
        st_r[...] = jnp.zeros_like(st_r)

    st_r[0:1, :] += jnp.sum(pre, axis=0, keepdims=True)
    st_r[1:2, :] += jnp.sum(pre * pre, axis=0, keepdims=True)


def _k1(agg, xd, cnt16, eaw, p):
    n = xd.shape[0]
    wep = jnp.pad(p['W_e'], ((0, 0), (0, 12)))
    wg1 = p['W_g'][:, :128]
    wg2 = p['W_g'][:, 128:]
    aux = jnp.zeros((8, 128), jnp.float32)
    aux = aux.at[0].set(p['b_l']).at[1].set(p['b_e']).at[2].set(p['b_g'])
    grid = (n // BN,)
    fixed = lambda i: (0, 0)
    blk = lambda shp: pl.BlockSpec(shp, fixed)
    pre, st = pl.pallas_call(
        _k1_body,
        grid=grid,
        in_specs=[
            pl.BlockSpec((BN, 128), lambda i: (i, 0)),
            pl.BlockSpec((BN, 128), lambda i: (i, 0)),
            pl.BlockSpec((BN, 16), lambda i: (i, 0)),
            pl.BlockSpec((BN, 16), lambda i: (i, 0)),
            blk((128, 128)), blk((128, 128)), blk((128, 16)),
            blk((128, 128)), blk((128, 128)), blk((8, 128)),
        ],
        out_specs=[
            pl.BlockSpec((BN, 128), lambda i: (i, 0)),
            pl.BlockSpec((8, 128), fixed),
        ],
        out_shape=[
            jax.ShapeDtypeStruct((n, 128), jnp.float32),
            jax.ShapeDtypeStruct((8, 128), jnp.float32),
        ],
    )(agg, xd, cnt16, eaw, p['W_l'], p['W_r'], wep, wg1, wg2, aux)
    return pre, st


def _make_k2_body(n, nrel, with_self, with_proj):
    fn = float(n)

    def body(*refs):
        k = 0
        acc = None
        for _ in range(nrel):
            pre_r, st_r, gb_r = refs[k], refs[k + 1], refs[k + 2]
            k += 3
            mu = st_r[0:1, :] / fn
            var = st_r[1:2, :] / fn - mu * mu
            rstd = lax.rsqrt(var + 1e-5)
            y = jnp.maximum(
                2.0 * ((pre_r[...] - mu) * rstd * gb_r[0:1, :] + gb_r[1:2, :]),
                0.0)
            acc = y if acc is None else acc + y
        if with_self:
            xg_r, ws_r, sb_r = refs[k], refs[k + 1], refs[k + 2]
            k += 3
            acc = acc + _dotT(xg_r[...], ws_r[...]) + sb_r[0:1, :]
        out = jnp.maximum(acc, 0.0)
        if with_proj:
            wo_r = refs[k]
            k += 1
            out = _dotT(out, wo_r[...])
        refs[-1][...] = out

    return body


def _k2(n, rels, selfterm=None, proj=None):
    grid = (n // BN,)
    fixed = lambda i: (0, 0)
    moving = pl.BlockSpec((BN, 128), lambda i: (i, 0))
    in_specs, args = [], []
    for pre, st, gb in rels:
        in_specs += [moving, pl.BlockSpec((8, 128), fixed),
                     pl.BlockSpec((8, 128), fixed)]
        args += [pre, st, gb]
    if selfterm is not None:
        xg, ws, sb = selfterm
        in_specs += [moving, pl.BlockSpec((128, 128), fixed),
                     pl.BlockSpec((8, 128), fixed)]
        args += [xg, ws, sb]
    if proj is not None:
        in_specs += [pl.BlockSpec((128, 128), fixed)]
        args += [proj]
    body = _make_k2_body(n, len(rels), selfterm is not None, proj is not None)
    return pl.pallas_call(
        body,
        grid=grid,
        in_specs=in_specs,
        out_specs=moving,
        out_shape=jax.ShapeDtypeStruct((n, 128), jnp.float32),
    )(*args)


def _gb(p):
    a = jnp.zeros((8, 128), jnp.float32)
    return a.at[0].set(p['gamma']).at[1].set(p['beta'])


def kernel(x_pfas, x_gw, x_sw, eas, params, eis):
    xcur = {'p': x_pfas, 'g': x_gw, 's': x_sw}
    nn = {k: int(v.shape[0]) for k, v in xcur.items()}

    rstat = {}
    for r in _RELS:
        row, col = eis[r][0], eis[r][1]
        n = nn[_SRCDST[r][1]]
        e = row.shape[0]
        cntf = jax.ops.segment_sum(jnp.ones((e,), jnp.float32), col,
                                   num_segments=n)
        win = jax.ops.segment_max(jnp.arange(e, dtype=jnp.int32), col,
                                  num_segments=n)
        winc = jnp.maximum(win, 0)
        ea_pad = jnp.pad(eas[r], ((0, 0), (0, 12)))
        eaw = ea_pad[winc]
        cnt16 = jnp.pad(cntf[:, None], ((0, 0), (0, 15)))
        rstat[r] = (cnt16, eaw)

    def hetero(P, xc):
        pre, st = {}, {}
        for r in _RELS:
            s, dk = _SRCDST[r]
            row, col = eis[r][0], eis[r][1]
            agg = jax.ops.segment_sum(xc[s][row], col,
                                      num_segments=nn[dk])
            cnt16, eaw = rstat[r]
            pre[r], st[r] = _k1(agg, xc[dk], cnt16, eaw, P[r])
        return pre, st

    P1 = params['l1']
    pre, st = hetero(P1, xcur)
    ws1 = P1['self']['W_l'] + P1['self']['W_r']
    sb1 = jnp.zeros((8, 128), jnp.float32).at[0].set(P1['self']['b_l'])
    xg1 = _k2(nn['g'], [(pre['pg'], st['pg'], _gb(P1['pg'])),
                        (pre['gg'], st['gg'], _gb(P1['gg']))],
              selfterm=(xcur['g'], ws1, sb1))
    xp1 = _k2(nn['p'], [(pre['gp'], st['gp'], _gb(P1['gp'])),
                        (pre['sp'], st['sp'], _gb(P1['sp']))])
    xs1 = _k2(nn['s'], [(pre['ps'], st['ps'], _gb(P1['ps']))])
    x1 = {'p': xp1, 'g': xg1, 's': xs1}

    P2 = params['l2']
    pre, st = hetero(P2, x1)
    ws2 = P2['self']['W_l'] + P2['self']['W_r']
    sb2 = jnp.zeros((8, 128), jnp.float32).at[0].set(P2['self']['b_l'])
    wog = jnp.zeros((128, 128), jnp.float32).at[0].set(params['W_out_gw'][0])
    wos = jnp.zeros((128, 128), jnp.float32).at[0].set(params['W_out_sw'][0])
    xg2 = _k2(nn['g'], [(pre['pg'], st['pg'], _gb(P2['pg'])),
                        (pre['gg'], st['gg'], _gb(P2['gg']))],
              selfterm=(x1['g'], ws2, sb2), proj=wog)
    xp2 = _k2(nn['p'], [(pre['gp'], st['gp'], _gb(P2['gp'])),
                        (pre['sp'], st['sp'], _gb(P2['sp']))])
    xs2 = _k2(nn['s'], [(pre['ps'], st['ps'], _gb(P2['ps']))], proj=wos)

    xg = xg2[:, :1] + params['b_out_gw']
    xs = xs2[:, :1] + params['b_out_sw']
    return xp2, xg, xs


# speedup vs baseline: 3.4708x; 1.5715x over previous
import functools

import jax
import jax.numpy as jnp
from jax import lax
from jax.experimental import pallas as pl
from jax.experimental.pallas import tpu as pltpu
from jax.experimental.pallas import tpu_sc as plsc

BN = 1000
EC = 28672


def _sc_agg(x_src, row3, col3, n_dst, steps):
    n_src = x_src.shape[0]
    x2d = x_src.reshape(n_src * 8, 16)
    nacc = (n_dst // 128 + 1) * 128
    dz = nacc // 16
    zfull, zrem = dz // 128, dz % 128
    mesh = plsc.VectorSubcoreMesh(core_axis_name="c", subcore_axis_name="s")

    ch = 7
    nch = steps // ch

    @functools.partial(
        pl.kernel,
        out_type=jax.ShapeDtypeStruct((nacc, 128), jnp.float32),
        mesh=mesh,
        compiler_params=pltpu.CompilerParams(use_tc_tiling_on_sc=False),
        scratch_types=[
            pltpu.VMEM_SHARED((nacc, 16), jnp.float32),
            pltpu.VMEM((2, ch, 128), jnp.int32),
            pltpu.VMEM((2, ch, 128), jnp.int32),
            pltpu.VMEM((2, 128), jnp.int32),
            pltpu.VMEM((128, 16), jnp.float32),
            pltpu.VMEM((128, 16), jnp.float32),
            pltpu.VMEM((128, 16), jnp.float32),
            pltpu.SemaphoreType.DMA,
            pltpu.SemaphoreType.DMA,
            pltpu.SemaphoreType.DMA,
            pltpu.SemaphoreType.DMA,
        ],
    )
    def k(x2_h, row_h, col_h, out_h, acc, rbuf, cbuf, gidx, dbuf0, dbuf1,
          zbuf, semg0, semg1, sempr, sempc):
        c = lax.axis_index("c")
        s = lax.axis_index("s")
        dbuf = (dbuf0, dbuf1)
        semg = (semg0, semg1)
        for i in range(128):
            zbuf[i] = jnp.zeros((16,), jnp.float32)

        def fill_issue(rows_row, par, f):
            for k8 in range(8):
                rv = rows_row[pl.ds(k8 * 16, 16)]
                gidx[par, pl.ds(k8 * 16, 16)] = rv * 8 + f
            pltpu.async_copy(x2_h.at[gidx.at[par]], dbuf[par], semg[par])

        def wait_g(par):
            pltpu.make_async_copy(x2_h.at[gidx.at[par]], dbuf[par],
                                  semg[par]).wait()

        for r in range(4):
            f = r * 2 + c
            base = s * dz

            def ms(i, _):
                pltpu.sync_copy(zbuf, acc.at[pl.ds(base + i * 128, 128)])
                return 0
            lax.fori_loop(0, zfull, ms, 0)
            if zrem:
                pltpu.sync_copy(zbuf.at[pl.ds(0, zrem)],
                                acc.at[pl.ds(base + zfull * 128, zrem)])
            plsc.subcore_barrier()

            pltpu.sync_copy(row_h.at[s, pl.ds(0, ch)], rbuf.at[0])
            pltpu.sync_copy(col_h.at[s, pl.ds(0, ch)], cbuf.at[0])
            fill_issue(rbuf.at[0, 0], 0, f)
            fill_issue(rbuf.at[0, 1], 1, f)

            def chunk_pair(i, _):
                for b in range(2):
                    cidx = 2 * i + b

                    @pl.when(cidx + 1 < nch)
                    def _():
                        nxt = (cidx + 1) * ch
                        pltpu.async_copy(row_h.at[s, pl.ds(nxt, ch)],
                                         rbuf.at[1 - b], sempr)
                        pltpu.async_copy(col_h.at[s, pl.ds(nxt, ch)],
                                         cbuf.at[1 - b], sempc)

                    for t in range(ch):
                        par = (b + t) % 2
                        if t == ch - 2:
                            @pl.when(cidx + 1 < nch)
                            def _():
                                pltpu.make_async_copy(
                                    row_h.at[s, pl.ds(0, ch)],
                                    rbuf.at[1 - b], sempr).wait()
                                pltpu.make_async_copy(
                                    col_h.at[s, pl.ds(0, ch)],
                                    cbuf.at[1 - b], sempc).wait()
                        wait_g(par)
                        pltpu.sync_copy(dbuf[par], acc.at[cbuf.at[b, t]],
                                        add=True)
                        if t < ch - 2:
                            fill_issue(rbuf.at[b, t + 2], par, f)
                        else:
                            @pl.when(cidx + 1 < nch)
                            def _():
                                fill_issue(rbuf.at[1 - b, t - (ch - 2)],
                                           par, f)
                return 0
            lax.fori_loop(0, nch // 2, chunk_pair, 0)
            plsc.subcore_barrier()
            pltpu.sync_copy(acc.at[pl.ds(s * dz, dz)],
                            out_h.at[pl.ds(s * dz, dz), pl.ds(f * 16, 16)])
            plsc.subcore_barrier()

    return k(x2d, row3, col3)

_RELS = ('pg', 'gg', 'gp', 'sp', 'ps')
_SRCDST = {'pg': ('p', 'g'), 'gg': ('g', 'g'), 'gp': ('g', 'p'),
           'sp': ('s', 'p'), 'ps': ('p', 's')}


def _dotT(x, w):
    return lax.dot_general(x, w, (((1,), (1,)), ((), ())),
                           preferred_element_type=jnp.float32)


def _k1_body(agg_r, xd_r, cnt_r, eaw_r, wl_r, wr_r, we_r, wg1_r, wg2_r,
             aux_r, pre_r, st_r):
    i = pl.program_id(0)
    cnt = cnt_r[:, 0:1]
    has = (cnt > 0.0).astype(jnp.float32)
    inv = 1.0 / jnp.maximum(cnt, 1.0)
    mean = agg_r[...] * inv
    P = _dotT(mean, wl_r[...]) + aux_r[0:1, :] + _dotT(xd_r[...], wr_r[...])
    E = _dotT(eaw_r[...], we_r[...]) + aux_r[1:2, :]
    G = jax.nn.sigmoid(_dotT(P, wg1_r[...]) + _dotT(E, wg2_r[...])
                       + aux_r[2:3, :])
    pre = P + has * (G * E)
    pre_r[...] = pre

    @pl.when(i == 0)
    def _():
        st_r[...] = jnp.zeros_like(st_r)

    st_r[0:1, :] += jnp.sum(pre, axis=0, keepdims=True)
    st_r[1:2, :] += jnp.sum(pre * pre, axis=0, keepdims=True)


def _k1(agg, xd, cnt16, eaw, p):
    n = xd.shape[0]
    wep = jnp.pad(p['W_e'], ((0, 0), (0, 12)))
    wg1 = p['W_g'][:, :128]
    wg2 = p['W_g'][:, 128:]
    aux = jnp.zeros((8, 128), jnp.float32)
    aux = aux.at[0].set(p['b_l']).at[1].set(p['b_e']).at[2].set(p['b_g'])
    grid = (n // BN,)
    fixed = lambda i: (0, 0)
    blk = lambda shp: pl.BlockSpec(shp, fixed)
    pre, st = pl.pallas_call(
        _k1_body,
        grid=grid,
        in_specs=[
            pl.BlockSpec((BN, 128), lambda i: (i, 0)),
            pl.BlockSpec((BN, 128), lambda i: (i, 0)),
            pl.BlockSpec((BN, 16), lambda i: (i, 0)),
            pl.BlockSpec((BN, 16), lambda i: (i, 0)),
            blk((128, 128)), blk((128, 128)), blk((128, 16)),
            blk((128, 128)), blk((128, 128)), blk((8, 128)),
        ],
        out_specs=[
            pl.BlockSpec((BN, 128), lambda i: (i, 0)),
            pl.BlockSpec((8, 128), fixed),
        ],
        out_shape=[
            jax.ShapeDtypeStruct((n, 128), jnp.float32),
            jax.ShapeDtypeStruct((8, 128), jnp.float32),
        ],
    )(agg, xd, cnt16, eaw, p['W_l'], p['W_r'], wep, wg1, wg2, aux)
    return pre, st


def _make_k2_body(n, nrel, with_self, with_proj):
    fn = float(n)

    def body(*refs):
        k = 0
        acc = None
        for _ in range(nrel):
            pre_r, st_r, gb_r = refs[k], refs[k + 1], refs[k + 2]
            k += 3
            mu = st_r[0:1, :] / fn
            var = st_r[1:2, :] / fn - mu * mu
            rstd = lax.rsqrt(var + 1e-5)
            y = jnp.maximum(
                2.0 * ((pre_r[...] - mu) * rstd * gb_r[0:1, :] + gb_r[1:2, :]),
                0.0)
            acc = y if acc is None else acc + y
        if with_self:
            xg_r, ws_r, sb_r = refs[k], refs[k + 1], refs[k + 2]
            k += 3
            acc = acc + _dotT(xg_r[...], ws_r[...]) + sb_r[0:1, :]
        out = jnp.maximum(acc, 0.0)
        if with_proj:
            wo_r = refs[k]
            k += 1
            out = _dotT(out, wo_r[...])
        refs[-1][...] = out

    return body


def _k2(n, rels, selfterm=None, proj=None):
    grid = (n // BN,)
    fixed = lambda i: (0, 0)
    moving = pl.BlockSpec((BN, 128), lambda i: (i, 0))
    in_specs, args = [], []
    for pre, st, gb in rels:
        in_specs += [moving, pl.BlockSpec((8, 128), fixed),
                     pl.BlockSpec((8, 128), fixed)]
        args += [pre, st, gb]
    if selfterm is not None:
        xg, ws, sb = selfterm
        in_specs += [moving, pl.BlockSpec((128, 128), fixed),
                     pl.BlockSpec((8, 128), fixed)]
        args += [xg, ws, sb]
    if proj is not None:
        in_specs += [pl.BlockSpec((128, 128), fixed)]
        args += [proj]
    body = _make_k2_body(n, len(rels), selfterm is not None, proj is not None)
    return pl.pallas_call(
        body,
        grid=grid,
        in_specs=in_specs,
        out_specs=moving,
        out_shape=jax.ShapeDtypeStruct((n, 128), jnp.float32),
    )(*args)


def _gb(p):
    a = jnp.zeros((8, 128), jnp.float32)
    return a.at[0].set(p['gamma']).at[1].set(p['beta'])


def kernel(x_pfas, x_gw, x_sw, eas, params, eis):
    xcur = {'p': x_pfas, 'g': x_gw, 's': x_sw}
    nn = {k: int(v.shape[0]) for k, v in xcur.items()}

    rstat = {}
    for r in _RELS:
        row, col = eis[r][0], eis[r][1]
        n = nn[_SRCDST[r][1]]
        e = row.shape[0]
        cntf = jax.ops.segment_sum(jnp.ones((e,), jnp.float32), col,
                                   num_segments=n)
        win = jax.ops.segment_max(jnp.arange(e, dtype=jnp.int32), col,
                                  num_segments=n)
        winc = jnp.maximum(win, 0)
        ea_pad = jnp.pad(eas[r], ((0, 0), (0, 12)))
        eaw = ea_pad[winc]
        cnt16 = jnp.pad(cntf[:, None], ((0, 0), (0, 15)))
        epad = -e % EC
        rowp = jnp.pad(row, (0, epad))
        colp = jnp.pad(col, (0, epad), constant_values=n)
        steps = (e + epad) // 2048
        row3 = rowp.reshape(16, steps, 128)
        col3 = colp.reshape(16, steps, 128)
        rstat[r] = (cnt16, eaw, row3, col3, steps)

    def hetero(P, xc):
        pre, st = {}, {}
        for r in _RELS:
            s, dk = _SRCDST[r]
            cnt16, eaw, row3, col3, steps = rstat[r]
            agg = _sc_agg(xc[s], row3, col3, nn[dk], steps)
            pre[r], st[r] = _k1(agg, xc[dk], cnt16, eaw, P[r])
        return pre, st

    P1 = params['l1']
    pre, st = hetero(P1, xcur)
    ws1 = P1['self']['W_l'] + P1['self']['W_r']
    sb1 = jnp.zeros((8, 128), jnp.float32).at[0].set(P1['self']['b_l'])
    xg1 = _k2(nn['g'], [(pre['pg'], st['pg'], _gb(P1['pg'])),
                        (pre['gg'], st['gg'], _gb(P1['gg']))],
              selfterm=(xcur['g'], ws1, sb1))
    xp1 = _k2(nn['p'], [(pre['gp'], st['gp'], _gb(P1['gp'])),
                        (pre['sp'], st['sp'], _gb(P1['sp']))])
    xs1 = _k2(nn['s'], [(pre['ps'], st['ps'], _gb(P1['ps']))])
    x1 = {'p': xp1, 'g': xg1, 's': xs1}

    P2 = params['l2']
    pre, st = hetero(P2, x1)
    ws2 = P2['self']['W_l'] + P2['self']['W_r']
    sb2 = jnp.zeros((8, 128), jnp.float32).at[0].set(P2['self']['b_l'])
    wog = jnp.zeros((128, 128), jnp.float32).at[0].set(params['W_out_gw'][0])
    wos = jnp.zeros((128, 128), jnp.float32).at[0].set(params['W_out_sw'][0])
    xg2 = _k2(nn['g'], [(pre['pg'], st['pg'], _gb(P2['pg'])),
                        (pre['gg'], st['gg'], _gb(P2['gg']))],
              selfterm=(x1['g'], ws2, sb2), proj=wog)
    xp2 = _k2(nn['p'], [(pre['gp'], st['gp'], _gb(P2['gp'])),
                        (pre['sp'], st['sp'], _gb(P2['sp']))])
    xs2 = _k2(nn['s'], [(pre['ps'], st['ps'], _gb(P2['ps']))], proj=wos)

    xg = xg2[:, :1] + params['b_out_gw']
    xs = xs2[:, :1] + params['b_out_sw']
    return xp2, xg, xs
